# baseline (speedup 1.0000x reference)
import functools

import jax
import jax.numpy as jnp
from jax.experimental import pallas as pl
from jax.experimental.pallas import tpu as pltpu

_EPS = 1e-5
_TILE_ROWS = 2048
_WAIT_ROWS = 128
_WAVE = 64


def _round_up(x: int, m: int) -> int:
    return (x + m - 1) // m * m


def _gather_ln_kernel(tile, n_waves, chunk, s_pad, manual_out,
                      ids_ref,
                      word_hbm,
                      pos_ref,
                      gamma_ref,
                      beta_ref,
                      out_ref,
                      tok_buf,
                      sems,
                      wsem):
    g = pl.program_id(0) * pl.num_programs(1) + pl.program_id(1)
    wave = tile // n_waves

    if tile % s_pad == 0:
        bpt = tile // s_pad
        b0 = g * bpt
        for i in range(tile):
            rid = ids_ref[b0 + i // s_pad, i % s_pad]
            pltpu.make_async_copy(word_hbm.at[pl.ds(rid, 1)],
                                  tok_buf.at[pl.ds(i, 1)],
                                  sems.at[i // wave]).start(priority=i & 1)
    else:
        n_sub = s_pad // tile
        b0 = g // n_sub
        s0 = (g % n_sub) * tile
        for i in range(tile):
            rid = ids_ref[b0, s0 + i]
            pltpu.make_async_copy(word_hbm.at[pl.ds(rid, 1)],
                                  tok_buf.at[pl.ds(i, 1)],
                                  sems.at[i // wave]).start()

    gamma = gamma_ref[...].reshape(1, gamma_ref.shape[-1])
    beta = beta_ref[...].reshape(1, beta_ref.shape[-1])

    for w in range(n_waves):
        pltpu.make_async_copy(word_hbm.at[pl.ds(0, wave)],
                              tok_buf.at[pl.ds(w * wave, wave)],
                              sems.at[w]).wait()
        for k in range(wave // chunk):
            r0 = w * wave + k * chunk
            rows = pl.ds(r0, chunk)
            if tile % s_pad == 0:
                ob, os = r0 // s_pad, r0 % s_pad
                pos_rows = pl.ds(os, chunk)
            else:
                ob, os = 0, r0
                pos_rows = pl.ds((g % (s_pad // tile)) * tile + r0, chunk)
            z = tok_buf[rows, :] + pos_ref[pos_rows, :]
            mean = jnp.mean(z, axis=-1, keepdims=True)
            c = z - mean
            var = jnp.mean(c * c, axis=-1, keepdims=True)
            res = c * jax.lax.rsqrt(var + _EPS) * gamma + beta
            if manual_out:
                tok_buf[rows, :] = res
                pltpu.make_async_copy(
                    tok_buf.at[rows],
                    out_ref.at[b0 + ob, pl.ds(os, chunk)],
                    wsem).start(priority=1)
            else:
                out_ref[ob, pl.ds(os, chunk), :] = res

    if manual_out:
        pltpu.make_async_copy(tok_buf.at[pl.ds(0, tile)],
                              tok_buf.at[pl.ds(0, tile)],
                              wsem).wait()


def kernel(input_ids, word_table, pos_table, gamma, beta):
    B, S = input_ids.shape
    V, H = word_table.shape

    s_pad = _round_up(S, 8)
    n_rows = B * s_pad
    tile = _TILE_ROWS
    while n_rows % tile != 0 or (tile % s_pad != 0 and s_pad % tile != 0):
        tile //= 2
    n_tiles = n_rows // tile

    def _pick_wave(divisor_of):
        for w in range(min(_WAVE, divisor_of), 7, -1):
            if w % 8 == 0 and divisor_of % w == 0:
                return w
        return 8
    wave = _pick_wave(s_pad if tile % s_pad == 0 else tile)
    wait_rows = max(wave, min(_WAIT_ROWS, tile))
    while tile % wait_rows != 0 or wait_rows % wave != 0:
        wait_rows //= 2
    n_waves = tile // wait_rows

    ids = input_ids if input_ids.dtype == jnp.int32 else input_ids.astype(jnp.int32)
    if s_pad != S:
        ids = jnp.pad(ids, ((0, 0), (0, s_pad - S)))
    pos = pos_table[:S] if pos_table.dtype == jnp.float32 else pos_table[:S].astype(jnp.float32)
    if s_pad != S:
        pos = jnp.pad(pos, ((0, s_pad - S), (0, 0)))

    n_cores = 2 if n_tiles % 2 == 0 else 1
    tiles_per_core = n_tiles // n_cores
    grid = (n_cores, tiles_per_core)

    def _tile_idx(c, t):
        return c * tiles_per_core + t

    manual_out = (tiles_per_core == 1) and (tile % s_pad == 0)

    if manual_out:
        out_spec = pl.BlockSpec(memory_space=pl.ANY)
    elif tile % s_pad == 0:
        bpt = tile // s_pad
        out_spec = pl.BlockSpec((bpt, s_pad, H),
                                lambda c, t, *_: (_tile_idx(c, t), 0, 0))
    else:
        n_sub = s_pad // tile
        out_spec = pl.BlockSpec((1, tile, H),
                                lambda c, t, *_: (_tile_idx(c, t) // n_sub,
                                                  _tile_idx(c, t) % n_sub, 0))

    kernel_fn = functools.partial(_gather_ln_kernel, tile, n_waves, wave,
                                  s_pad, manual_out)
    out = pl.pallas_call(
        kernel_fn,
        out_shape=jax.ShapeDtypeStruct((B, s_pad, H), jnp.float32),
        grid_spec=pltpu.PrefetchScalarGridSpec(
            num_scalar_prefetch=1,
            grid=grid,
            in_specs=[
                pl.BlockSpec(memory_space=pl.ANY),
                pl.BlockSpec((s_pad, H), lambda c, t, *_: (0, 0)),
                pl.BlockSpec((H,), lambda c, t, *_: (0,)),
                pl.BlockSpec((H,), lambda c, t, *_: (0,)),
            ],
            out_specs=out_spec,
            scratch_shapes=[
                pltpu.VMEM((tile, H), jnp.float32),
                pltpu.SemaphoreType.DMA((n_waves,)),
                pltpu.SemaphoreType.DMA,
            ]),
        compiler_params=pltpu.CompilerParams(
            dimension_semantics=("parallel", "arbitrary"),
            disable_bounds_checks=True,
            vmem_limit_bytes=64 << 20),
    )(ids, word_table, pos, gamma, beta)

    return out if s_pad == S else out[:, :S, :]


# speedup vs baseline: 1.1175x; 1.1175x over previous
import functools

import jax
import jax.numpy as jnp
from jax.experimental import pallas as pl
from jax.experimental.pallas import tpu as pltpu

_EPS = 1e-5
_TILE_ROWS = 2048
_WAIT_ROWS = 256
_WAVE = 64


def _round_up(x: int, m: int) -> int:
    return (x + m - 1) // m * m


def _gather_ln_kernel(tile, n_waves, chunk, s_pad, manual_out,
                      ids_ref,
                      word_hbm,
                      pos_ref,
                      gamma_ref,
                      beta_ref,
                      out_ref,
                      tok_buf,
                      sems,
                      wsem):
    g = pl.program_id(0) * pl.num_programs(1) + pl.program_id(1)
    wave = tile // n_waves

    if tile % s_pad == 0:
        bpt = tile // s_pad
        b0 = g * bpt
        for i in range(tile):
            rid = ids_ref[b0 + i // s_pad, i % s_pad]
            pltpu.make_async_copy(word_hbm.at[pl.ds(rid, 1)],
                                  tok_buf.at[pl.ds(i, 1)],
                                  sems.at[i // wave]).start(priority=i & 1)
    else:
        n_sub = s_pad // tile
        b0 = g // n_sub
        s0 = (g % n_sub) * tile
        for i in range(tile):
            rid = ids_ref[b0, s0 + i]
            pltpu.make_async_copy(word_hbm.at[pl.ds(rid, 1)],
                                  tok_buf.at[pl.ds(i, 1)],
                                  sems.at[i // wave]).start()

    gamma = gamma_ref[...].reshape(1, gamma_ref.shape[-1])
    beta = beta_ref[...].reshape(1, beta_ref.shape[-1])

    for w in range(n_waves):
        pltpu.make_async_copy(word_hbm.at[pl.ds(0, wave)],
                              tok_buf.at[pl.ds(w * wave, wave)],
                              sems.at[w]).wait()
        for k in range(wave // chunk):
            r0 = w * wave + k * chunk
            rows = pl.ds(r0, chunk)
            if tile % s_pad == 0:
                ob, os = r0 // s_pad, r0 % s_pad
                pos_rows = pl.ds(os, chunk)
            else:
                ob, os = 0, r0
                pos_rows = pl.ds((g % (s_pad // tile)) * tile + r0, chunk)
            z = tok_buf[rows, :] + pos_ref[pos_rows, :]
            mean = jnp.mean(z, axis=-1, keepdims=True)
            c = z - mean
            var = jnp.mean(c * c, axis=-1, keepdims=True)
            res = c * jax.lax.rsqrt(var + _EPS) * gamma + beta
            if manual_out:
                tok_buf[rows, :] = res
                pltpu.make_async_copy(
                    tok_buf.at[rows],
                    out_ref.at[b0 + ob, pl.ds(os, chunk)],
                    wsem).start(priority=1)
            else:
                out_ref[ob, pl.ds(os, chunk), :] = res

    if manual_out:
        pltpu.make_async_copy(tok_buf.at[pl.ds(0, tile)],
                              tok_buf.at[pl.ds(0, tile)],
                              wsem).wait()


def kernel(input_ids, word_table, pos_table, gamma, beta):
    B, S = input_ids.shape
    V, H = word_table.shape

    s_pad = _round_up(S, 8)
    n_rows = B * s_pad
    tile = _TILE_ROWS
    while n_rows % tile != 0 or (tile % s_pad != 0 and s_pad % tile != 0):
        tile //= 2
    n_tiles = n_rows // tile

    def _pick_wave(divisor_of):
        for w in range(min(_WAVE, divisor_of), 7, -1):
            if w % 8 == 0 and divisor_of % w == 0:
                return w
        return 8
    wave = _pick_wave(s_pad if tile % s_pad == 0 else tile)
    wait_rows = max(wave, min(_WAIT_ROWS, tile))
    while tile % wait_rows != 0 or wait_rows % wave != 0:
        wait_rows //= 2
    n_waves = tile // wait_rows

    ids = input_ids if input_ids.dtype == jnp.int32 else input_ids.astype(jnp.int32)
    if s_pad != S:
        ids = jnp.pad(ids, ((0, 0), (0, s_pad - S)))
    pos = pos_table[:S] if pos_table.dtype == jnp.float32 else pos_table[:S].astype(jnp.float32)
    if s_pad != S:
        pos = jnp.pad(pos, ((0, s_pad - S), (0, 0)))

    n_cores = 2 if n_tiles % 2 == 0 else 1
    tiles_per_core = n_tiles // n_cores
    grid = (n_cores, tiles_per_core)

    def _tile_idx(c, t):
        return c * tiles_per_core + t

    manual_out = (tiles_per_core == 1) and (tile % s_pad == 0)

    if manual_out:
        out_spec = pl.BlockSpec(memory_space=pl.ANY)
    elif tile % s_pad == 0:
        bpt = tile // s_pad
        out_spec = pl.BlockSpec((bpt, s_pad, H),
                                lambda c, t, *_: (_tile_idx(c, t), 0, 0))
    else:
        n_sub = s_pad // tile
        out_spec = pl.BlockSpec((1, tile, H),
                                lambda c, t, *_: (_tile_idx(c, t) // n_sub,
                                                  _tile_idx(c, t) % n_sub, 0))

    kernel_fn = functools.partial(_gather_ln_kernel, tile, n_waves, wave,
                                  s_pad, manual_out)
    out = pl.pallas_call(
        kernel_fn,
        out_shape=jax.ShapeDtypeStruct((B, s_pad, H), jnp.float32),
        grid_spec=pltpu.PrefetchScalarGridSpec(
            num_scalar_prefetch=1,
            grid=grid,
            in_specs=[
                pl.BlockSpec(memory_space=pl.ANY),
                pl.BlockSpec((s_pad, H), lambda c, t, *_: (0, 0)),
                pl.BlockSpec((H,), lambda c, t, *_: (0,)),
                pl.BlockSpec((H,), lambda c, t, *_: (0,)),
            ],
            out_specs=out_spec,
            scratch_shapes=[
                pltpu.VMEM((tile, H), jnp.float32),
                pltpu.SemaphoreType.DMA((n_waves,)),
                pltpu.SemaphoreType.DMA,
            ]),
        compiler_params=pltpu.CompilerParams(
            dimension_semantics=("parallel", "arbitrary"),
            disable_bounds_checks=True,
            vmem_limit_bytes=64 << 20),
    )(ids, word_table, pos, gamma, beta)

    return out if s_pad == S else out[:, :S, :]


# speedup vs baseline: 1.1178x; 1.0003x over previous
import functools

import jax
import jax.numpy as jnp
from jax.experimental import pallas as pl
from jax.experimental.pallas import tpu as pltpu

_EPS = 1e-5
_TILE_ROWS = 2048
_WAIT_ROWS = 256
_WAVE = 64


def _round_up(x: int, m: int) -> int:
    return (x + m - 1) // m * m


def _gather_ln_kernel(tile, n_waves, chunk, s_pad, manual_out,
                      ids_ref,
                      word_hbm,
                      pos_ref,
                      gamma_ref,
                      beta_ref,
                      out_ref,
                      tok_buf,
                      sems,
                      wsem):
    g = pl.program_id(0) * pl.num_programs(1) + pl.program_id(1)
    wave = tile // n_waves

    if tile % s_pad == 0:
        bpt = tile // s_pad
        b0 = g * bpt
        for i in range(tile):
            rid = ids_ref[b0 + i // s_pad, i % s_pad]
            pltpu.make_async_copy(word_hbm.at[pl.ds(rid, 1)],
                                  tok_buf.at[pl.ds(i, 1)],
                                  sems.at[i // wave]).start(priority=i & 1)
    else:
        n_sub = s_pad // tile
        b0 = g // n_sub
        s0 = (g % n_sub) * tile
        for i in range(tile):
            rid = ids_ref[b0, s0 + i]
            pltpu.make_async_copy(word_hbm.at[pl.ds(rid, 1)],
                                  tok_buf.at[pl.ds(i, 1)],
                                  sems.at[i // wave]).start(priority=i & 1)

    gamma = gamma_ref[...].reshape(1, gamma_ref.shape[-1])
    beta = beta_ref[...].reshape(1, beta_ref.shape[-1])

    for w in range(n_waves):
        pltpu.make_async_copy(word_hbm.at[pl.ds(0, wave)],
                              tok_buf.at[pl.ds(w * wave, wave)],
                              sems.at[w]).wait()
        for k in range(wave // chunk):
            r0 = w * wave + k * chunk
            rows = pl.ds(r0, chunk)
            if tile % s_pad == 0:
                ob, os = r0 // s_pad, r0 % s_pad
                pos_rows = pl.ds(os, chunk)
            else:
                ob, os = 0, r0
                pos_rows = pl.ds((g % (s_pad // tile)) * tile + r0, chunk)
            z = tok_buf[rows, :] + pos_ref[pos_rows, :]
            mean = jnp.mean(z, axis=-1, keepdims=True)
            c = z - mean
            var = jnp.mean(c * c, axis=-1, keepdims=True)
            res = c * jax.lax.rsqrt(var + _EPS) * gamma + beta
            if manual_out:
                tok_buf[rows, :] = res
                pltpu.make_async_copy(
                    tok_buf.at[rows],
                    out_ref.at[b0 + ob, pl.ds(os, chunk)],
                    wsem).start(priority=1)
            else:
                out_ref[ob, pl.ds(os, chunk), :] = res

    if manual_out:
        pltpu.make_async_copy(tok_buf.at[pl.ds(0, tile)],
                              tok_buf.at[pl.ds(0, tile)],
                              wsem).wait()


def kernel(input_ids, word_table, pos_table, gamma, beta):
    B, S = input_ids.shape
    V, H = word_table.shape

    s_pad = _round_up(S, 8)
    n_rows = B * s_pad
    tile = _TILE_ROWS
    while n_rows % tile != 0 or (tile % s_pad != 0 and s_pad % tile != 0):
        tile //= 2
    n_tiles = n_rows // tile

    def _pick_wave(divisor_of):
        for w in range(min(_WAVE, divisor_of), 7, -1):
            if w % 8 == 0 and divisor_of % w == 0:
                return w
        return 8
    wave = _pick_wave(s_pad if tile % s_pad == 0 else tile)
    wait_rows = max(wave, min(_WAIT_ROWS, tile))
    while tile % wait_rows != 0 or wait_rows % wave != 0:
        wait_rows //= 2
    n_waves = tile // wait_rows

    ids = input_ids if input_ids.dtype == jnp.int32 else input_ids.astype(jnp.int32)
    if s_pad != S:
        ids = jnp.pad(ids, ((0, 0), (0, s_pad - S)))
    pos = pos_table[:S] if pos_table.dtype == jnp.float32 else pos_table[:S].astype(jnp.float32)
    if s_pad != S:
        pos = jnp.pad(pos, ((0, s_pad - S), (0, 0)))

    n_cores = 2 if n_tiles % 2 == 0 else 1
    tiles_per_core = n_tiles // n_cores
    grid = (n_cores, tiles_per_core)

    def _tile_idx(c, t):
        return c * tiles_per_core + t

    manual_out = (tiles_per_core == 1) and (tile % s_pad == 0)

    if manual_out:
        out_spec = pl.BlockSpec(memory_space=pl.ANY)
    elif tile % s_pad == 0:
        bpt = tile // s_pad
        out_spec = pl.BlockSpec((bpt, s_pad, H),
                                lambda c, t, *_: (_tile_idx(c, t), 0, 0))
    else:
        n_sub = s_pad // tile
        out_spec = pl.BlockSpec((1, tile, H),
                                lambda c, t, *_: (_tile_idx(c, t) // n_sub,
                                                  _tile_idx(c, t) % n_sub, 0))

    kernel_fn = functools.partial(_gather_ln_kernel, tile, n_waves, wave,
                                  s_pad, manual_out)
    out = pl.pallas_call(
        kernel_fn,
        out_shape=jax.ShapeDtypeStruct((B, s_pad, H), jnp.float32),
        grid_spec=pltpu.PrefetchScalarGridSpec(
            num_scalar_prefetch=1,
            grid=grid,
            in_specs=[
                pl.BlockSpec(memory_space=pl.ANY),
                pl.BlockSpec((s_pad, H), lambda c, t, *_: (0, 0)),
                pl.BlockSpec((H,), lambda c, t, *_: (0,)),
                pl.BlockSpec((H,), lambda c, t, *_: (0,)),
            ],
            out_specs=out_spec,
            scratch_shapes=[
                pltpu.VMEM((tile, H), jnp.float32),
                pltpu.SemaphoreType.DMA((n_waves,)),
                pltpu.SemaphoreType.DMA,
            ]),
        compiler_params=pltpu.CompilerParams(
            dimension_semantics=("parallel", "arbitrary"),
            disable_bounds_checks=True,
            vmem_limit_bytes=64 << 20),
    )(ids, word_table, pos, gamma, beta)

    return out if s_pad == S else out[:, :S, :]
